# Initial kernel scaffold; baseline (speedup 1.0000x reference)
import jax
import jax.numpy as jnp
from jax.experimental import pallas as pl


def kernel(x, o):
    raise NotImplementedError("write your pallas kernel here")


# speedup vs baseline: 2.6074x; 2.6074x over previous
import jax
import jax.numpy as jnp
from jax.experimental import pallas as pl
from jax.experimental.pallas import tpu as pltpu

_TAU = 1.0
_B, _N, _M = 64, 512, 512


def _fused_body(x_ref, o_ref, out_ref):
    e = jnp.exp(-x_ref[0] * (1.0 / _TAU)) * o_ref[0]
    m = jnp.max(e)
    rows = jax.lax.broadcasted_iota(jnp.int32, (_N, _M), 0)
    cols = jax.lax.broadcasted_iota(jnp.int32, (_N, _M), 1)
    flat = rows * _M + cols
    idx = jnp.min(jnp.where(e == m, flat, jnp.int32(2**31 - 1)))
    out_ref[0] = (flat == idx).astype(jnp.float32)


def kernel(x, o):
    return pl.pallas_call(
        _fused_body,
        grid=(_B,),
        in_specs=[
            pl.BlockSpec((1, _N, _M), lambda b: (b, 0, 0)),
            pl.BlockSpec((1, _N, _M), lambda b: (b, 0, 0)),
        ],
        out_specs=pl.BlockSpec((1, _N, _M), lambda b: (b, 0, 0)),
        out_shape=jax.ShapeDtypeStruct((_B, _N, _M), jnp.float32),
    )(x, o)
